# baseline (speedup 1.0000x reference)
import functools

import jax
import jax.numpy as jnp
from jax import lax
from jax.experimental import pallas as pl
from jax.experimental.pallas import tpu as pltpu
from jax.experimental.pallas import tpu_sc as plsc

_N = 10000
_E = 320000
_D = 128
_H = _D // 2
_EPS = 1e-3

_TBLK = 2000


def _tables_body(atom_ref, mi_ref, mj_ref, hbc_ref, pi_ref, pj_ref):
    a = atom_ref[...]
    half_bc = hbc_ref[0:1, :]
    pi = jnp.dot(a, mi_ref[...], preferred_element_type=jnp.float32) + half_bc
    pj = jnp.dot(a, mj_ref[...], preferred_element_type=jnp.float32) + half_bc
    pi_ref[...] = pi.astype(jnp.bfloat16)
    pj_ref[...] = pj.astype(jnp.bfloat16)


def _project_tables(atom, mi, mj, half_bc):
    return pl.pallas_call(
        _tables_body,
        grid=(_N // _TBLK,),
        in_specs=[
            pl.BlockSpec((_TBLK, _D), lambda i: (i, 0)),
            pl.BlockSpec((_D, _D), lambda i: (0, 0)),
            pl.BlockSpec((_D, _D), lambda i: (0, 0)),
            pl.BlockSpec((8, _D), lambda i: (0, 0)),
        ],
        out_specs=[
            pl.BlockSpec((_TBLK, _D), lambda i: (i, 0)),
            pl.BlockSpec((_TBLK, _D), lambda i: (i, 0)),
        ],
        out_shape=[
            jax.ShapeDtypeStruct((_N, _D), jnp.bfloat16),
            jax.ShapeDtypeStruct((_N, _D), jnp.bfloat16),
        ],
    )(atom, mi, mj, half_bc)


_NC = 2
_NS = 16
_NW = _NC * _NS
_PER_W = _E // _NW
_CHUNK = 200
_NCHUNK = _PER_W // _CHUNK


def _make_gather_sum():
    mesh = plsc.VectorSubcoreMesh(core_axis_name="c", subcore_axis_name="s")

    @functools.partial(
        pl.kernel,
        mesh=mesh,
        out_type=jax.ShapeDtypeStruct((_E, _H), jnp.float32),
        compiler_params=pltpu.CompilerParams(
            use_tc_tiling_on_sc=False, needs_layout_passes=False),
        scratch_types=[
            pltpu.VMEM((_CHUNK,), jnp.int32),
            pltpu.VMEM((_CHUNK,), jnp.int32),
            pltpu.VMEM((_CHUNK,), jnp.int32),
            pltpu.VMEM((_CHUNK,), jnp.int32),
            pltpu.VMEM((_CHUNK, _H), jnp.float32),
            pltpu.VMEM((_CHUNK, _H), jnp.float32),
            pltpu.VMEM((_CHUNK, _H), jnp.float32),
            pltpu.VMEM((_CHUNK, _H), jnp.float32),
            pltpu.VMEM((_CHUNK, _H), jnp.float32),
            pltpu.VMEM((_CHUNK, _H), jnp.float32),
            pltpu.SemaphoreType.DMA,
            pltpu.SemaphoreType.DMA,
            pltpu.SemaphoreType.DMA,
            pltpu.SemaphoreType.DMA,
            pltpu.SemaphoreType.DMA,
            pltpu.SemaphoreType.DMA,
        ],
    )
    def gather_sum(pi_hbm, pj_hbm, ii_hbm, jj_hbm, out_hbm,
                   ic0, ic1, jc0, jc1, ba0, ba1, bb0, bb1, bo0, bo1,
                   sa0, sa1, sb0, sb1, so0, so1):
        wid = lax.axis_index("s") * _NC + lax.axis_index("c")
        base = wid * _PER_W
        idxi = (ic0, ic1)
        idxj = (jc0, jc1)
        bufa = (ba0, ba1)
        bufb = (bb0, bb1)
        bufo = (bo0, bo1)
        sga = (sa0, sa1)
        sgb = (sb0, sb1)
        sso = (so0, so1)

        def prime(g, b):
            sl = pl.ds(base + g * _CHUNK, _CHUNK)
            pltpu.sync_copy(ii_hbm.at[sl], idxi[b])
            pltpu.sync_copy(jj_hbm.at[sl], idxj[b])
            pltpu.make_async_copy(pi_hbm.at[idxi[b]], bufa[b], sga[b]).start()
            pltpu.make_async_copy(pj_hbm.at[idxj[b]], bufb[b], sgb[b]).start()

        def wait_gathers(b):
            pltpu.make_async_copy(pi_hbm.at[idxi[b]], bufa[b], sga[b]).wait()
            pltpu.make_async_copy(pj_hbm.at[idxj[b]], bufb[b], sgb[b]).wait()

        def store_chunk(g, b):
            rows = pl.ds(base + g * _CHUNK, _CHUNK)
            return pltpu.make_async_copy(bufo[b], out_hbm.at[rows], sso[b])

        def sum_chunk(b):
            def add_body(r, c2):
                for l in range(_H // 16):
                    s = pl.ds(l * 16, 16)
                    x = plsc.bitcast(bufa[b][r, s], jnp.bfloat16)
                    y = plsc.bitcast(bufb[b][r, s], jnp.bfloat16)
                    bufo[b][r, s] = plsc.bitcast(x + y, jnp.float32)
                return c2
            lax.fori_loop(0, _CHUNK, add_body, 0)

        prime(0, 0)
        prime(1, 1)
        for g in (0, 1):
            b = g
            wait_gathers(b)
            sum_chunk(b)
            store_chunk(g, b).start()
            prime(g + 2, b)

        def steady(g2, carry):
            for b in range(2):
                g = g2 * 2 + b
                wait_gathers(b)
                store_chunk(g - 2, b).wait()
                sum_chunk(b)
                store_chunk(g, b).start()
                prime(g + 2, b)
            return carry

        lax.fori_loop(1, _NCHUNK // 2 - 1, steady, 0)

        for g in (_NCHUNK - 2, _NCHUNK - 1):
            b = g % 2
            wait_gathers(b)
            store_chunk(g - 2, b).wait()
            sum_chunk(b)
            store_chunk(g, b).start()
        for b in range(2):
            store_chunk(_NCHUNK - 2 + b, b).wait()

    return gather_sum


_EBLK = 4000


def _edge_body(bond_ref, gath_ref, wci_ref, out_ref):
    b = bond_ref[...]
    out_ref[...] = gath_ref[...].astype(jnp.float32) + jnp.dot(
        b, wci_ref[...], preferred_element_type=jnp.float32)


def _edge_update(bond, gath_bf, wci):
    return pl.pallas_call(
        _edge_body,
        grid=(_E // _EBLK,),
        in_specs=[
            pl.BlockSpec((_EBLK, _D), lambda i: (i, 0)),
            pl.BlockSpec((_EBLK, _D), lambda i: (i, 0)),
            pl.BlockSpec((_D, _D), lambda i: (0, 0)),
        ],
        out_specs=pl.BlockSpec((_EBLK, _D), lambda i: (i, 0)),
        out_shape=jax.ShapeDtypeStruct((_E, _D), jnp.float32),
    )(bond, gath_bf, wci)


def kernel(atom_embedding, bond_embedding, indices_i, indices_j,
           W1, b1, gamma1, beta1, mean1, var1,
           W2, b2, gamma2, beta2, mean2, var2):
    s1 = gamma1 / jnp.sqrt(var1 + _EPS)
    t1 = beta1 - mean1 * s1
    s2 = gamma2 / jnp.sqrt(var2 + _EPS)
    t2 = beta2 - mean2 * s2
    wf = (s1[:, None] * W2) * s2[None, :]
    bf = (t1 @ W2 + b2) * s2 + t2
    mi = W1[:_D] @ wf
    wc = W1[_D:2 * _D] @ wf
    mj = W1[2 * _D:] @ wf
    bc = b1 @ wf + bf
    wci = wc + jnp.eye(_D, dtype=jnp.float32)
    half_bc = jnp.broadcast_to(0.5 * bc, (8, _D))

    pi_bf, pj_bf = _project_tables(atom_embedding, mi, mj, half_bc)
    pi_pk = lax.bitcast_convert_type(pi_bf.reshape(_N, _H, 2), jnp.float32)
    pj_pk = lax.bitcast_convert_type(pj_bf.reshape(_N, _H, 2), jnp.float32)
    gath_pk = _make_gather_sum()(pi_pk, pj_pk, indices_i, indices_j)
    gath_bf = lax.bitcast_convert_type(gath_pk, jnp.bfloat16).reshape(_E, _D)
    return _edge_update(bond_embedding, gath_bf, wci)


# speedup vs baseline: 2.7015x; 2.7015x over previous
import functools

import jax
import numpy as np
import jax.numpy as jnp
from jax import lax
from jax.experimental import pallas as pl
from jax.experimental.pallas import tpu as pltpu
from jax.experimental.pallas import tpu_sc as plsc

_N = 10000
_E = 320000
_D = 128
_H = _D // 2
_EPS = 1e-3

_PERM = np.concatenate([np.arange(0, _D, 2), np.arange(1, _D, 2)])
_PMAT_NP = np.zeros((_D, _D), dtype=np.float32)
for _k in range(_H):
    _PMAT_NP[_k, 2 * _k] = 1.0
    _PMAT_NP[_H + _k, 2 * _k + 1] = 1.0

_TBLK = 2000


def _pack_rows(h):
    u = lax.bitcast_convert_type(h, jnp.uint32)
    r = (u + jnp.uint32(0x7FFF) + ((u >> 16) & jnp.uint32(1))) >> 16
    w = r[:, :_H] | (r[:, _H:] << 16)
    return lax.bitcast_convert_type(w, jnp.float32)


def _tables_body(atom_ref, mi_ref, mj_ref, hbc_ref, pi_ref, pj_ref):
    a = atom_ref[...]
    half_bc = hbc_ref[0:1, :]
    pi = jnp.dot(a, mi_ref[...], preferred_element_type=jnp.float32) + half_bc
    pj = jnp.dot(a, mj_ref[...], preferred_element_type=jnp.float32) + half_bc
    pi_ref[...] = _pack_rows(pi)
    pj_ref[...] = _pack_rows(pj)


def _project_tables(atom, mi, mj, half_bc):
    return pl.pallas_call(
        _tables_body,
        grid=(_N // _TBLK,),
        in_specs=[
            pl.BlockSpec((_TBLK, _D), lambda i: (i, 0)),
            pl.BlockSpec((_D, _D), lambda i: (0, 0)),
            pl.BlockSpec((_D, _D), lambda i: (0, 0)),
            pl.BlockSpec((8, _D), lambda i: (0, 0)),
        ],
        out_specs=[
            pl.BlockSpec((_TBLK, _H), lambda i: (i, 0)),
            pl.BlockSpec((_TBLK, _H), lambda i: (i, 0)),
        ],
        out_shape=[
            jax.ShapeDtypeStruct((_N, _H), jnp.float32),
            jax.ShapeDtypeStruct((_N, _H), jnp.float32),
        ],
    )(atom, mi, mj, half_bc)


_NC = 2
_NS = 16
_NW = _NC * _NS
_PER_W = _E // _NW
_CHUNK = 200
_NCHUNK = _PER_W // _CHUNK


def _make_gather_sum():
    mesh = plsc.VectorSubcoreMesh(core_axis_name="c", subcore_axis_name="s")

    @functools.partial(
        pl.kernel,
        mesh=mesh,
        out_type=jax.ShapeDtypeStruct((_E, _H), jnp.float32),
        compiler_params=pltpu.CompilerParams(
            use_tc_tiling_on_sc=False, needs_layout_passes=False),
        scratch_types=[
            pltpu.VMEM((_CHUNK,), jnp.int32),
            pltpu.VMEM((_CHUNK,), jnp.int32),
            pltpu.VMEM((_CHUNK,), jnp.int32),
            pltpu.VMEM((_CHUNK,), jnp.int32),
            pltpu.VMEM((_CHUNK, _H), jnp.float32),
            pltpu.VMEM((_CHUNK, _H), jnp.float32),
            pltpu.VMEM((_CHUNK, _H), jnp.float32),
            pltpu.VMEM((_CHUNK, _H), jnp.float32),
            pltpu.VMEM((_CHUNK, _H), jnp.float32),
            pltpu.VMEM((_CHUNK, _H), jnp.float32),
            pltpu.SemaphoreType.DMA,
            pltpu.SemaphoreType.DMA,
            pltpu.SemaphoreType.DMA,
            pltpu.SemaphoreType.DMA,
            pltpu.SemaphoreType.DMA,
            pltpu.SemaphoreType.DMA,
        ],
    )
    def gather_sum(pi_hbm, pj_hbm, ii_hbm, jj_hbm, out_hbm,
                   ic0, ic1, jc0, jc1, ba0, ba1, bb0, bb1, bo0, bo1,
                   sa0, sa1, sb0, sb1, so0, so1):
        wid = lax.axis_index("s") * _NC + lax.axis_index("c")
        base = wid * _PER_W
        idxi = (ic0, ic1)
        idxj = (jc0, jc1)
        bufa = (ba0, ba1)
        bufb = (bb0, bb1)
        bufo = (bo0, bo1)
        sga = (sa0, sa1)
        sgb = (sb0, sb1)
        sso = (so0, so1)

        def prime(g, b):
            sl = pl.ds(base + g * _CHUNK, _CHUNK)
            pltpu.sync_copy(ii_hbm.at[sl], idxi[b])
            pltpu.sync_copy(jj_hbm.at[sl], idxj[b])
            pltpu.make_async_copy(pi_hbm.at[idxi[b]], bufa[b], sga[b]).start()
            pltpu.make_async_copy(pj_hbm.at[idxj[b]], bufb[b], sgb[b]).start()

        def wait_gathers(b):
            pltpu.make_async_copy(pi_hbm.at[idxi[b]], bufa[b], sga[b]).wait()
            pltpu.make_async_copy(pj_hbm.at[idxj[b]], bufb[b], sgb[b]).wait()

        def store_chunk(g, b):
            rows = pl.ds(base + g * _CHUNK, _CHUNK)
            return pltpu.make_async_copy(bufo[b], out_hbm.at[rows], sso[b])

        def sum_chunk(b):
            def add_body(r, c2):
                for l in range(_H // 16):
                    s = pl.ds(l * 16, 16)
                    x = plsc.bitcast(bufa[b][r, s], jnp.bfloat16)
                    y = plsc.bitcast(bufb[b][r, s], jnp.bfloat16)
                    bufo[b][r, s] = plsc.bitcast(x + y, jnp.float32)
                return c2
            lax.fori_loop(0, _CHUNK, add_body, 0)

        prime(0, 0)
        prime(1, 1)
        for g in (0, 1):
            b = g
            wait_gathers(b)
            sum_chunk(b)
            store_chunk(g, b).start()
            prime(g + 2, b)

        def steady(g2, carry):
            for b in range(2):
                g = g2 * 2 + b
                wait_gathers(b)
                store_chunk(g - 2, b).wait()
                sum_chunk(b)
                store_chunk(g, b).start()
                prime(g + 2, b)
            return carry

        lax.fori_loop(1, _NCHUNK // 2 - 1, steady, 0)

        for g in (_NCHUNK - 2, _NCHUNK - 1):
            b = g % 2
            wait_gathers(b)
            store_chunk(g - 2, b).wait()
            sum_chunk(b)
            store_chunk(g, b).start()
        for b in range(2):
            store_chunk(_NCHUNK - 2 + b, b).wait()

    return gather_sum


_EBLK = 4000


def _edge_body(bond_ref, gath_ref, wcip_ref, p_ref, out_ref):
    w = lax.bitcast_convert_type(gath_ref[...], jnp.uint32)
    even = lax.bitcast_convert_type(w << 16, jnp.float32)
    odd = lax.bitcast_convert_type(w & jnp.uint32(0xFFFF0000), jnp.float32)
    ocat = jnp.concatenate([even, odd], axis=1)
    acc = ocat + jnp.dot(bond_ref[...], wcip_ref[...],
                         preferred_element_type=jnp.float32)
    out_ref[...] = jnp.dot(acc, p_ref[...], preferred_element_type=jnp.float32)


def _edge_update(bond, gath_pk, wcip, pmat):
    return pl.pallas_call(
        _edge_body,
        grid=(_E // _EBLK,),
        in_specs=[
            pl.BlockSpec((_EBLK, _D), lambda i: (i, 0)),
            pl.BlockSpec((_EBLK, _H), lambda i: (i, 0)),
            pl.BlockSpec((_D, _D), lambda i: (0, 0)),
            pl.BlockSpec((_D, _D), lambda i: (0, 0)),
        ],
        out_specs=pl.BlockSpec((_EBLK, _D), lambda i: (i, 0)),
        out_shape=jax.ShapeDtypeStruct((_E, _D), jnp.float32),
    )(bond, gath_pk, wcip, pmat)


def kernel(atom_embedding, bond_embedding, indices_i, indices_j,
           W1, b1, gamma1, beta1, mean1, var1,
           W2, b2, gamma2, beta2, mean2, var2):
    s1 = gamma1 / jnp.sqrt(var1 + _EPS)
    t1 = beta1 - mean1 * s1
    s2 = gamma2 / jnp.sqrt(var2 + _EPS)
    t2 = beta2 - mean2 * s2
    wf = (s1[:, None] * W2) * s2[None, :]
    bf = (t1 @ W2 + b2) * s2 + t2
    mi = W1[:_D] @ wf
    wc = W1[_D:2 * _D] @ wf
    mj = W1[2 * _D:] @ wf
    bc = b1 @ wf + bf
    wci = wc + jnp.eye(_D, dtype=jnp.float32)

    mi_p = mi[:, _PERM]
    mj_p = mj[:, _PERM]
    wcip = wci[:, _PERM]
    half_bc = jnp.broadcast_to((0.5 * bc)[_PERM], (8, _D))

    pi_pk, pj_pk = _project_tables(atom_embedding, mi_p, mj_p, half_bc)
    gath_pk = _make_gather_sum()(pi_pk, pj_pk, indices_i, indices_j)
    return _edge_update(bond_embedding, gath_pk, wcip, jnp.asarray(_PMAT_NP))


# speedup vs baseline: 2.8145x; 1.0418x over previous
import functools

import jax
import numpy as np
import jax.numpy as jnp
from jax import lax
from jax.experimental import pallas as pl
from jax.experimental.pallas import tpu as pltpu
from jax.experimental.pallas import tpu_sc as plsc

_N = 10000
_E = 320000
_D = 128
_H = _D // 2
_EPS = 1e-3

_PERM = np.concatenate([np.arange(0, _D, 2), np.arange(1, _D, 2)])
_PMAT_NP = np.zeros((_D, _D), dtype=np.float32)
for _k in range(_H):
    _PMAT_NP[_k, 2 * _k] = 1.0
    _PMAT_NP[_H + _k, 2 * _k + 1] = 1.0

_TBLK = 2000


def _pack_rows(h):
    u = lax.bitcast_convert_type(h, jnp.uint32)
    r = (u + jnp.uint32(0x7FFF) + ((u >> 16) & jnp.uint32(1))) >> 16
    w = r[:, :_H] | (r[:, _H:] << 16)
    return lax.bitcast_convert_type(w, jnp.float32)


def _tables_body(atom_ref, mi_ref, mj_ref, hbc_ref, pi_ref, pj_ref):
    a = atom_ref[...]
    half_bc = hbc_ref[0:1, :]
    pi = jnp.dot(a, mi_ref[...], preferred_element_type=jnp.float32) + half_bc
    pj = jnp.dot(a, mj_ref[...], preferred_element_type=jnp.float32) + half_bc
    pi_ref[...] = _pack_rows(pi)
    pj_ref[...] = _pack_rows(pj)


def _project_tables(atom, mi, mj, half_bc):
    return pl.pallas_call(
        _tables_body,
        grid=(_N // _TBLK,),
        in_specs=[
            pl.BlockSpec((_TBLK, _D), lambda i: (i, 0)),
            pl.BlockSpec((_D, _D), lambda i: (0, 0)),
            pl.BlockSpec((_D, _D), lambda i: (0, 0)),
            pl.BlockSpec((8, _D), lambda i: (0, 0)),
        ],
        out_specs=[
            pl.BlockSpec((_TBLK, _H), lambda i: (i, 0)),
            pl.BlockSpec((_TBLK, _H), lambda i: (i, 0)),
        ],
        out_shape=[
            jax.ShapeDtypeStruct((_N, _H), jnp.float32),
            jax.ShapeDtypeStruct((_N, _H), jnp.float32),
        ],
    )(atom, mi, mj, half_bc)


_NC = 2
_NS = 16
_NW = _NC * _NS
_NSEG = 5
_SEG = _E // _NSEG
_PER_W = _SEG // _NW
_CHUNK = 200
_NCHUNK = _PER_W // _CHUNK


def _make_gather_sum(seg_base):
    mesh = plsc.VectorSubcoreMesh(core_axis_name="c", subcore_axis_name="s")

    @functools.partial(
        pl.kernel,
        mesh=mesh,
        out_type=jax.ShapeDtypeStruct((_SEG, _H), jnp.float32),
        compiler_params=pltpu.CompilerParams(
            use_tc_tiling_on_sc=False, needs_layout_passes=False),
        scratch_types=[
            pltpu.VMEM((_CHUNK,), jnp.int32),
            pltpu.VMEM((_CHUNK,), jnp.int32),
            pltpu.VMEM((_CHUNK,), jnp.int32),
            pltpu.VMEM((_CHUNK,), jnp.int32),
            pltpu.VMEM((_CHUNK, _H), jnp.float32),
            pltpu.VMEM((_CHUNK, _H), jnp.float32),
            pltpu.VMEM((_CHUNK, _H), jnp.float32),
            pltpu.VMEM((_CHUNK, _H), jnp.float32),
            pltpu.VMEM((_CHUNK, _H), jnp.float32),
            pltpu.VMEM((_CHUNK, _H), jnp.float32),
            pltpu.SemaphoreType.DMA,
            pltpu.SemaphoreType.DMA,
            pltpu.SemaphoreType.DMA,
            pltpu.SemaphoreType.DMA,
            pltpu.SemaphoreType.DMA,
            pltpu.SemaphoreType.DMA,
        ],
    )
    def gather_sum(pi_hbm, pj_hbm, ii_hbm, jj_hbm, out_hbm,
                   ic0, ic1, jc0, jc1, ba0, ba1, bb0, bb1, bo0, bo1,
                   sa0, sa1, sb0, sb1, so0, so1):
        wid = lax.axis_index("s") * _NC + lax.axis_index("c")
        base = wid * _PER_W
        ibase = seg_base + base
        idxi = (ic0, ic1)
        idxj = (jc0, jc1)
        bufa = (ba0, ba1)
        bufb = (bb0, bb1)
        bufo = (bo0, bo1)
        sga = (sa0, sa1)
        sgb = (sb0, sb1)
        sso = (so0, so1)

        def prime(g, b):
            sl = pl.ds(ibase + g * _CHUNK, _CHUNK)
            pltpu.sync_copy(ii_hbm.at[sl], idxi[b])
            pltpu.sync_copy(jj_hbm.at[sl], idxj[b])
            pltpu.make_async_copy(pi_hbm.at[idxi[b]], bufa[b], sga[b]).start()
            pltpu.make_async_copy(pj_hbm.at[idxj[b]], bufb[b], sgb[b]).start()

        def wait_gathers(b):
            pltpu.make_async_copy(pi_hbm.at[idxi[b]], bufa[b], sga[b]).wait()
            pltpu.make_async_copy(pj_hbm.at[idxj[b]], bufb[b], sgb[b]).wait()

        def store_chunk(g, b):
            rows = pl.ds(base + g * _CHUNK, _CHUNK)
            return pltpu.make_async_copy(bufo[b], out_hbm.at[rows], sso[b])

        def sum_chunk(b):
            def add_body(r, c2):
                for l in range(_H // 16):
                    s = pl.ds(l * 16, 16)
                    x = plsc.bitcast(bufa[b][r, s], jnp.bfloat16)
                    y = plsc.bitcast(bufb[b][r, s], jnp.bfloat16)
                    bufo[b][r, s] = plsc.bitcast(x + y, jnp.float32)
                return c2
            lax.fori_loop(0, _CHUNK, add_body, 0)

        prime(0, 0)
        prime(1, 1)
        for g in (0, 1):
            b = g
            wait_gathers(b)
            sum_chunk(b)
            store_chunk(g, b).start()
            prime(g + 2, b)

        def steady(g2, carry):
            for b in range(2):
                g = g2 * 2 + b
                wait_gathers(b)
                store_chunk(g - 2, b).wait()
                sum_chunk(b)
                store_chunk(g, b).start()
                prime(g + 2, b)
            return carry

        lax.fori_loop(1, _NCHUNK // 2 - 1, steady, 0)

        for g in (_NCHUNK - 2, _NCHUNK - 1):
            b = g % 2
            wait_gathers(b)
            store_chunk(g - 2, b).wait()
            sum_chunk(b)
            store_chunk(g, b).start()
        for b in range(2):
            store_chunk(_NCHUNK - 2 + b, b).wait()

    return gather_sum


_EBLK = 4000


def _edge_body(bond_ref, gath_ref, wcip_ref, p_ref, out_ref):
    w = lax.bitcast_convert_type(gath_ref[...], jnp.uint32)
    even = lax.bitcast_convert_type(w << 16, jnp.float32)
    odd = lax.bitcast_convert_type(w & jnp.uint32(0xFFFF0000), jnp.float32)
    ocat = jnp.concatenate([even, odd], axis=1)
    acc = ocat + jnp.dot(bond_ref[...], wcip_ref[...],
                         preferred_element_type=jnp.float32)
    out_ref[...] = jnp.dot(acc, p_ref[...], preferred_element_type=jnp.float32)


_SBLK = _SEG // _EBLK


def _edge_body_first(bond_ref, gath_ref, wcip_ref, p_ref, out_ref):
    _edge_body(bond_ref, gath_ref, wcip_ref, p_ref, out_ref)


def _edge_body_chained(bond_ref, gath_ref, wcip_ref, p_ref, prev_ref, out_ref):
    del prev_ref
    _edge_body(bond_ref, gath_ref, wcip_ref, p_ref, out_ref)


def _edge_update_seg(k, bond, gath_pk, wcip, pmat, prev):
    blk0 = k * _SBLK
    common_in = [
        pl.BlockSpec((_EBLK, _D), lambda i, b=blk0: (b + i, 0)),
        pl.BlockSpec((_EBLK, _H), lambda i: (i, 0)),
        pl.BlockSpec((_D, _D), lambda i: (0, 0)),
        pl.BlockSpec((_D, _D), lambda i: (0, 0)),
    ]
    out_spec = pl.BlockSpec((_EBLK, _D), lambda i, b=blk0: (b + i, 0))
    out_shape = jax.ShapeDtypeStruct((_E, _D), jnp.float32)
    if prev is None:
        return pl.pallas_call(
            _edge_body_first,
            grid=(_SBLK,),
            in_specs=common_in,
            out_specs=out_spec,
            out_shape=out_shape,
        )(bond, gath_pk, wcip, pmat)
    return pl.pallas_call(
        _edge_body_chained,
        grid=(_SBLK,),
        in_specs=common_in + [pl.BlockSpec(memory_space=pl.ANY)],
        out_specs=out_spec,
        out_shape=out_shape,
        input_output_aliases={4: 0},
    )(bond, gath_pk, wcip, pmat, prev)


def kernel(atom_embedding, bond_embedding, indices_i, indices_j,
           W1, b1, gamma1, beta1, mean1, var1,
           W2, b2, gamma2, beta2, mean2, var2):
    s1 = gamma1 / jnp.sqrt(var1 + _EPS)
    t1 = beta1 - mean1 * s1
    s2 = gamma2 / jnp.sqrt(var2 + _EPS)
    t2 = beta2 - mean2 * s2
    wf = (s1[:, None] * W2) * s2[None, :]
    bf = (t1 @ W2 + b2) * s2 + t2
    mi = W1[:_D] @ wf
    wc = W1[_D:2 * _D] @ wf
    mj = W1[2 * _D:] @ wf
    bc = b1 @ wf + bf
    wci = wc + jnp.eye(_D, dtype=jnp.float32)

    mi_p = mi[:, _PERM]
    mj_p = mj[:, _PERM]
    wcip = wci[:, _PERM]
    half_bc = jnp.broadcast_to((0.5 * bc)[_PERM], (8, _D))

    pi_pk, pj_pk = _project_tables(atom_embedding, mi_p, mj_p, half_bc)
    pmat = jnp.asarray(_PMAT_NP)
    out = None
    for k in range(_NSEG):
        gath_k = _make_gather_sum(k * _SEG)(pi_pk, pj_pk, indices_i, indices_j)
        out = _edge_update_seg(k, bond_embedding, gath_k, wcip, pmat, out)
    return out
